# Initial kernel scaffold; baseline (speedup 1.0000x reference)
import jax
import jax.numpy as jnp
from jax.experimental import pallas as pl


def kernel(y_true, y_pred, B):
    raise NotImplementedError("write your pallas kernel here")


# speedup vs baseline: 5.0766x; 5.0766x over previous
import functools

import jax
import jax.numpy as jnp
from jax import lax
from jax.experimental import pallas as pl
from jax.experimental.pallas import tpu as pltpu
from jax.experimental.pallas import tpu_sc as plsc

VOCAB = 24
LANES = 16
N_CORES = 2
N_SUBCORES = 16
N_WORKERS = N_CORES * N_SUBCORES
ROWS_PER_STREAM = 128
STREAMS_PER_CHUNK = 8
CHUNK = ROWS_PER_STREAM * STREAMS_PER_CHUNK


def _make_kernel(chunks_per_worker: int):
  mesh = plsc.VectorSubcoreMesh(core_axis_name="c", subcore_axis_name="s")

  @functools.partial(
      pl.kernel,
      out_type=jax.ShapeDtypeStruct((N_WORKERS, LANES), jnp.float32),
      mesh=mesh,
      scratch_types=[
          pltpu.VMEM((STREAMS_PER_CHUNK, ROWS_PER_STREAM, VOCAB), jnp.float32),
          pltpu.VMEM((STREAMS_PER_CHUNK, ROWS_PER_STREAM), jnp.int32),
          pltpu.VMEM((VOCAB, VOCAB), jnp.float32),
          pltpu.VMEM((VOCAB, VOCAB), jnp.float32),
          pltpu.VMEM((LANES,), jnp.float32),
          pltpu.VMEM_SHARED((N_SUBCORES, VOCAB, VOCAB), jnp.float32),
      ],
      compiler_params=pltpu.CompilerParams(use_tc_tiling_on_sc=False),
  )
  def blosum_kernel(yt_hbm, yp_hbm, b_hbm, out_hbm,
                    pred_v, idx_v, stage_v, b_v, acc_v, s_sh):
    cid = lax.axis_index("c")
    sid = lax.axis_index("s")
    wid = sid * N_CORES + cid

    pltpu.sync_copy(b_hbm, b_v)
    zeros = jnp.zeros((LANES,), jnp.float32)
    for r in range(VOCAB):
      stage_v[r, pl.ds(0, LANES)] = zeros
      stage_v[r, pl.ds(VOCAB - LANES, LANES)] = zeros
    pltpu.sync_copy(stage_v, s_sh.at[sid])

    def body(i, carry):
      c = wid * chunks_per_worker + i
      pltpu.sync_copy(yt_hbm.at[c], idx_v)
      pltpu.sync_copy(yp_hbm.at[c], pred_v)
      for j in range(STREAMS_PER_CHUNK):
        pltpu.sync_copy(pred_v.at[j], s_sh.at[sid].at[idx_v.at[j]], add=True)
      return carry

    lax.fori_loop(0, chunks_per_worker, body, 0)

    pltpu.sync_copy(s_sh.at[sid], stage_v)
    lane = lax.iota(jnp.int32, LANES)
    hi_mask = lane >= (2 * LANES - VOCAB)
    acc = jnp.zeros((LANES,), jnp.float32)
    for r in range(VOCAB):
      acc = acc + stage_v[r, pl.ds(0, LANES)] * b_v[r, pl.ds(0, LANES)]
      hi = stage_v[r, pl.ds(VOCAB - LANES, LANES)] * b_v[r, pl.ds(VOCAB - LANES, LANES)]
      acc = acc + jnp.where(hi_mask, hi, zeros)
    acc_v[...] = acc
    pltpu.sync_copy(acc_v, out_hbm.at[wid])

  return blosum_kernel


@jax.jit
def kernel(y_true, y_pred, B):
  n_pos = y_true.shape[0] * y_true.shape[1]
  n_chunks = n_pos // CHUNK
  chunks_per_worker = n_chunks // N_WORKERS
  yt = y_true.reshape(n_chunks, STREAMS_PER_CHUNK, ROWS_PER_STREAM)
  yp = y_pred.reshape(n_chunks, STREAMS_PER_CHUNK, ROWS_PER_STREAM, VOCAB)
  partials = _make_kernel(chunks_per_worker)(yt, yp, B)
  return jnp.sum(partials)


# speedup vs baseline: 5.3560x; 1.0550x over previous
import functools

import jax
import jax.numpy as jnp
from jax import lax
from jax.experimental import pallas as pl
from jax.experimental.pallas import tpu as pltpu
from jax.experimental.pallas import tpu_sc as plsc

VOCAB = 24
LANES = 16
N_CORES = 2
N_SUBCORES = 16
N_WORKERS = N_CORES * N_SUBCORES
ROWS_PER_STREAM = 128
STREAMS_PER_CHUNK = 8
CHUNK = ROWS_PER_STREAM * STREAMS_PER_CHUNK
NBUF = 4
NGEN = 2


def _make_kernel(chunks_per_worker: int):
  mesh = plsc.VectorSubcoreMesh(core_axis_name="c", subcore_axis_name="s")

  scratch = (
      [pltpu.VMEM((STREAMS_PER_CHUNK, ROWS_PER_STREAM, VOCAB), jnp.float32)] * NBUF
      + [pltpu.VMEM((STREAMS_PER_CHUNK, ROWS_PER_STREAM), jnp.int32)] * NBUF
      + [
          pltpu.VMEM((VOCAB, VOCAB), jnp.float32),
          pltpu.VMEM((VOCAB, VOCAB), jnp.float32),
          pltpu.VMEM((LANES,), jnp.float32),
          pltpu.VMEM_SHARED(
              (N_SUBCORES, NGEN, STREAMS_PER_CHUNK, VOCAB, VOCAB), jnp.float32),
      ]
      + [pltpu.SemaphoreType.DMA] * (2 * NBUF + NGEN)
  )

  @functools.partial(
      pl.kernel,
      out_type=jax.ShapeDtypeStruct((N_WORKERS, LANES), jnp.float32),
      mesh=mesh,
      scratch_types=scratch,
      compiler_params=pltpu.CompilerParams(use_tc_tiling_on_sc=False),
  )
  def blosum_kernel(yt_hbm, yp_hbm, b_hbm, out_hbm, *refs):
    pred_b = refs[0:NBUF]
    idx_b = refs[NBUF:2 * NBUF]
    stage_v, b_v, acc_v, s_sh = refs[2 * NBUF:2 * NBUF + 4]
    sems = refs[2 * NBUF + 4:]
    sem_pred = sems[0:NBUF]
    sem_idx = sems[NBUF:2 * NBUF]
    sem_sc = sems[2 * NBUF:2 * NBUF + NGEN]

    cid = lax.axis_index("c")
    sid = lax.axis_index("s")
    wid = sid * N_CORES + cid
    base = wid * chunks_per_worker
    my_s = s_sh.at[sid]

    pltpu.sync_copy(b_hbm, b_v)
    zeros = jnp.zeros((LANES,), jnp.float32)
    for r in range(VOCAB):
      stage_v[r, pl.ds(0, LANES)] = zeros
      stage_v[r, pl.ds(VOCAB - LANES, LANES)] = zeros
    for g in range(NGEN):
      for j in range(STREAMS_PER_CHUNK):
        pltpu.sync_copy(stage_v, my_s.at[g].at[j])

    pend_in = [None] * NBUF
    pend_sc = [None] * chunks_per_worker

    def start(c):
      b = c % NBUF
      pend_in[b] = (
          pltpu.async_copy(yt_hbm.at[base + c], idx_b[b], sem_idx[b]),
          pltpu.async_copy(yp_hbm.at[base + c], pred_b[b], sem_pred[b]),
      )

    for c in range(min(NBUF, chunks_per_worker)):
      start(c)

    for c in range(chunks_per_worker):
      b = c % NBUF
      for d in pend_in[b]:
        d.wait()
      if c >= NGEN:
        for d in pend_sc[c - NGEN]:
          d.wait()
        pend_sc[c - NGEN] = None
        if c - NGEN + NBUF < chunks_per_worker:
          start(c - NGEN + NBUF)
      g = c % NGEN
      pend_sc[c] = [
          pltpu.async_copy(
              pred_b[b].at[j], my_s.at[g].at[j].at[idx_b[b].at[j]],
              sem_sc[g], add=True)
          for j in range(STREAMS_PER_CHUNK)
      ]

    for c in range(chunks_per_worker):
      if pend_sc[c] is not None:
        for d in pend_sc[c]:
          d.wait()

    lane = lax.iota(jnp.int32, LANES)
    hi_mask = lane >= (2 * LANES - VOCAB)
    acc = jnp.zeros((LANES,), jnp.float32)
    for g in range(NGEN):
      for j in range(STREAMS_PER_CHUNK):
        pltpu.sync_copy(my_s.at[g].at[j], stage_v)
        for r in range(VOCAB):
          acc = acc + stage_v[r, pl.ds(0, LANES)] * b_v[r, pl.ds(0, LANES)]
          hi = (stage_v[r, pl.ds(VOCAB - LANES, LANES)]
                * b_v[r, pl.ds(VOCAB - LANES, LANES)])
          acc = acc + jnp.where(hi_mask, hi, zeros)
    acc_v[...] = acc
    pltpu.sync_copy(acc_v, out_hbm.at[wid])

  return blosum_kernel


@jax.jit
def kernel(y_true, y_pred, B):
  n_pos = y_true.shape[0] * y_true.shape[1]
  n_chunks = n_pos // CHUNK
  chunks_per_worker = n_chunks // N_WORKERS
  yt = y_true.reshape(n_chunks, STREAMS_PER_CHUNK, ROWS_PER_STREAM)
  yp = y_pred.reshape(n_chunks, STREAMS_PER_CHUNK, ROWS_PER_STREAM, VOCAB)
  partials = _make_kernel(chunks_per_worker)(yt, yp, B)
  return jnp.sum(partials)


# speedup vs baseline: 5.3634x; 1.0014x over previous
import functools

import jax
import jax.numpy as jnp
from jax import lax
from jax.experimental import pallas as pl
from jax.experimental.pallas import tpu as pltpu
from jax.experimental.pallas import tpu_sc as plsc

VOCAB = 24
LANES = 16
N_CORES = 2
N_SUBCORES = 16
N_WORKERS = N_CORES * N_SUBCORES
ROWS_PER_STREAM = 128
STREAMS_PER_CHUNK = 8
CHUNK = ROWS_PER_STREAM * STREAMS_PER_CHUNK
NBUF = 4
NGEN = 2


def _make_kernel(chunks_per_worker: int):
  mesh = plsc.VectorSubcoreMesh(core_axis_name="c", subcore_axis_name="s")

  scratch = (
      [pltpu.VMEM((CHUNK, VOCAB), jnp.float32)] * NBUF
      + [pltpu.VMEM((STREAMS_PER_CHUNK, ROWS_PER_STREAM), jnp.int32)] * NBUF
      + [
          pltpu.VMEM((VOCAB, VOCAB), jnp.float32),
          pltpu.VMEM((VOCAB, VOCAB), jnp.float32),
          pltpu.VMEM((LANES,), jnp.float32),
          pltpu.VMEM_SHARED(
              (N_SUBCORES, NGEN, STREAMS_PER_CHUNK, VOCAB, VOCAB), jnp.float32),
      ]
      + [pltpu.SemaphoreType.DMA] * (2 * NBUF + NGEN)
  )

  @functools.partial(
      pl.kernel,
      out_type=jax.ShapeDtypeStruct((N_WORKERS, LANES), jnp.float32),
      mesh=mesh,
      scratch_types=scratch,
      compiler_params=pltpu.CompilerParams(use_tc_tiling_on_sc=False),
  )
  def blosum_kernel(yt_hbm, yp_hbm, b_hbm, out_hbm, *refs):
    pred_b = refs[0:NBUF]
    idx_b = refs[NBUF:2 * NBUF]
    stage_v, b_v, acc_v, s_sh = refs[2 * NBUF:2 * NBUF + 4]
    sems = refs[2 * NBUF + 4:]
    sem_pred = sems[0:NBUF]
    sem_idx = sems[NBUF:2 * NBUF]
    sem_sc = sems[2 * NBUF:2 * NBUF + NGEN]

    cid = lax.axis_index("c")
    sid = lax.axis_index("s")
    wid = sid * N_CORES + cid
    base = wid * chunks_per_worker
    my_s = s_sh.at[sid]

    pltpu.sync_copy(b_hbm, b_v)
    zeros = jnp.zeros((LANES,), jnp.float32)
    for r in range(VOCAB):
      stage_v[r, pl.ds(0, LANES)] = zeros
      stage_v[r, pl.ds(VOCAB - LANES, LANES)] = zeros
    for g in range(NGEN):
      for j in range(STREAMS_PER_CHUNK):
        pltpu.sync_copy(stage_v, my_s.at[g].at[j])

    pend_in = [None] * NBUF
    pend_sc = [None] * chunks_per_worker

    def start(c):
      b = c % NBUF
      pend_in[b] = (
          pltpu.async_copy(yt_hbm.at[base + c], idx_b[b], sem_idx[b]),
          pltpu.async_copy(yp_hbm.at[base + c], pred_b[b], sem_pred[b]),
      )

    for c in range(min(NBUF, chunks_per_worker)):
      start(c)

    for c in range(chunks_per_worker):
      b = c % NBUF
      for d in pend_in[b]:
        d.wait()
      if c >= NGEN:
        for d in pend_sc[c - NGEN]:
          d.wait()
        pend_sc[c - NGEN] = None
        if c - NGEN + NBUF < chunks_per_worker:
          start(c - NGEN + NBUF)
      g = c % NGEN
      pend_sc[c] = [
          pltpu.async_copy(
              pred_b[b].at[pl.ds(j * ROWS_PER_STREAM, ROWS_PER_STREAM)],
              my_s.at[g].at[j].at[idx_b[b].at[j]],
              sem_sc[g], add=True)
          for j in range(STREAMS_PER_CHUNK)
      ]

    for c in range(chunks_per_worker):
      if pend_sc[c] is not None:
        for d in pend_sc[c]:
          d.wait()

    lane = lax.iota(jnp.int32, LANES)
    hi_mask = lane >= (2 * LANES - VOCAB)
    acc = jnp.zeros((LANES,), jnp.float32)
    for g in range(NGEN):
      for j in range(STREAMS_PER_CHUNK):
        pltpu.sync_copy(my_s.at[g].at[j], stage_v)
        for r in range(VOCAB):
          acc = acc + stage_v[r, pl.ds(0, LANES)] * b_v[r, pl.ds(0, LANES)]
          hi = (stage_v[r, pl.ds(VOCAB - LANES, LANES)]
                * b_v[r, pl.ds(VOCAB - LANES, LANES)])
          acc = acc + jnp.where(hi_mask, hi, zeros)
    acc_v[...] = acc
    pltpu.sync_copy(acc_v, out_hbm.at[wid])

  return blosum_kernel


@jax.jit
def kernel(y_true, y_pred, B):
  n_pos = y_true.shape[0] * y_true.shape[1]
  n_chunks = n_pos // CHUNK
  chunks_per_worker = n_chunks // N_WORKERS
  yt = y_true.reshape(n_chunks, STREAMS_PER_CHUNK, ROWS_PER_STREAM)
  yp = y_pred.reshape(n_chunks, CHUNK, VOCAB)
  partials = _make_kernel(chunks_per_worker)(yt, yp, B)
  return jnp.sum(partials)


# speedup vs baseline: 13.5826x; 2.5325x over previous
import functools

import jax
import jax.numpy as jnp
from jax import lax
from jax.experimental import pallas as pl
from jax.experimental.pallas import tpu as pltpu
from jax.experimental.pallas import tpu_sc as plsc

VOCAB = 24
LANES = 16
N_CORES = 2
N_SUBCORES = 16
N_WORKERS = N_CORES * N_SUBCORES
BGROUP = 8
SBLK = 2048
LANE_STRIDE = 577
ACC_WORDS = LANE_STRIDE * LANES


def _make_kernel(batch: int, seq: int):
  n_squarters = N_WORKERS // (batch // BGROUP)
  assert n_squarters * SBLK <= seq
  mesh = plsc.VectorSubcoreMesh(core_axis_name="c", subcore_axis_name="s")

  scratch = (
      [pltpu.VMEM((VOCAB, SBLK), jnp.float32)] * 2
      + [
          pltpu.VMEM((BGROUP, SBLK), jnp.int32),
          pltpu.VMEM((ACC_WORDS,), jnp.float32),
          pltpu.VMEM((VOCAB * VOCAB,), jnp.float32),
          pltpu.VMEM((LANES,), jnp.float32),
      ]
      + [pltpu.SemaphoreType.DMA] * 3
  )

  @functools.partial(
      pl.kernel,
      out_type=jax.ShapeDtypeStruct((N_WORKERS * LANES,), jnp.float32),
      mesh=mesh,
      scratch_types=scratch,
      compiler_params=pltpu.CompilerParams(
          use_tc_tiling_on_sc=True, needs_layout_passes=False),
  )
  def blosum_kernel(yt_hbm, yp_hbm, b_hbm, out_hbm,
                    pred0, pred1, idx_v, s_v, b_v, acc_v,
                    sem0, sem1, sem_i):
    pred_b = (pred0, pred1)
    sems = (sem0, sem1)
    cid = lax.axis_index("c")
    sid = lax.axis_index("s")
    wid = sid * N_CORES + cid
    bg = wid // n_squarters
    s0 = (wid % n_squarters) * SBLK

    d_idx = pltpu.async_copy(
        yt_hbm.at[pl.ds(bg * BGROUP, BGROUP), pl.ds(s0, SBLK)], idx_v, sem_i)
    pltpu.sync_copy(b_hbm, b_v)

    zeros = jnp.zeros((LANES,), jnp.float32)
    for i in range(ACC_WORDS // LANES):
      s_v[pl.ds(i * LANES, LANES)] = zeros

    lane_off = lax.iota(jnp.int32, LANES) * LANE_STRIDE

    def start(r, buf):
      return pltpu.async_copy(
          yp_hbm.at[bg * BGROUP + r].at[:, pl.ds(s0, SBLK)],
          pred_b[buf], sems[buf])

    pend = [start(0, 0), start(1, 1)]
    d_idx.wait()

    for r in range(BGROUP):
      buf = r % 2
      pend[buf].wait()
      pv = pred_b[buf]

      def sb_body(sb, carry):
        k = idx_v[r, pl.ds(sb * LANES, LANES)]
        k24 = k * VOCAB + lane_off
        for v in range(VOCAB):
          data = pv[v, pl.ds(sb * LANES, LANES)]
          plsc.addupdate_scatter(s_v, [k24 + v], data)
        return carry

      lax.fori_loop(0, SBLK // LANES, sb_body, 0)
      if r + 2 < BGROUP:
        pend[buf] = start(r + 2, buf)

    acc = jnp.zeros((LANES,), jnp.float32)
    for i in range(VOCAB * VOCAB // LANES):
      bv = b_v[pl.ds(i * LANES, LANES)]
      for l in range(LANES):
        acc = acc + s_v[pl.ds(l * LANE_STRIDE + i * LANES, LANES)] * bv
    acc_v[...] = acc
    pltpu.sync_copy(acc_v, out_hbm.at[pl.ds(wid * LANES, LANES)])

  return blosum_kernel


@jax.jit
def kernel(y_true, y_pred, B):
  batch, seq = y_true.shape
  yp_t = jnp.transpose(y_pred, (0, 2, 1))
  partials = _make_kernel(batch, seq)(y_true, yp_t, B.reshape(-1))
  return jnp.sum(partials)


# speedup vs baseline: 25.9469x; 1.9103x over previous
import functools

import jax
import jax.numpy as jnp
from jax import lax
from jax.experimental import pallas as pl
from jax.experimental.pallas import tpu as pltpu
from jax.experimental.pallas import tpu_sc as plsc

VOCAB = 24
LANES = 16
N_CORES = 2
N_SUBCORES = 16
N_WORKERS = N_CORES * N_SUBCORES
BGROUP = 8
SBLK = 2048
LANE_STRIDE = 577
REP_WORDS = LANE_STRIDE * LANES


def _make_kernel(batch: int, seq: int):
  n_sblocks = N_WORKERS // (batch // BGROUP)
  assert n_sblocks * SBLK <= seq
  mesh = plsc.VectorSubcoreMesh(core_axis_name="c", subcore_axis_name="s")

  scratch = (
      [pltpu.VMEM((VOCAB, SBLK), jnp.float32)] * 2
      + [
          pltpu.VMEM((BGROUP, SBLK), jnp.int32),
          pltpu.VMEM((VOCAB * VOCAB,), jnp.float32),
          pltpu.VMEM((REP_WORDS,), jnp.float32),
          pltpu.VMEM((LANES,), jnp.float32),
      ]
      + [pltpu.SemaphoreType.DMA] * 3
  )

  @functools.partial(
      pl.kernel,
      out_type=jax.ShapeDtypeStruct((N_WORKERS * LANES,), jnp.float32),
      mesh=mesh,
      scratch_types=scratch,
      compiler_params=pltpu.CompilerParams(
          use_tc_tiling_on_sc=True, needs_layout_passes=False),
  )
  def blosum_kernel(yt_hbm, yp_hbm, b_hbm, out_hbm,
                    pred0, pred1, idx_v, b_v, brep_v, acc_v,
                    sem0, sem1, sem_i):
    pred_b = (pred0, pred1)
    sems = (sem0, sem1)
    cid = lax.axis_index("c")
    sid = lax.axis_index("s")
    wid = sid * N_CORES + cid
    bg = wid // n_sblocks
    s0 = (wid % n_sblocks) * SBLK

    d_idx = pltpu.async_copy(
        yt_hbm.at[pl.ds(bg * BGROUP, BGROUP), pl.ds(s0, SBLK)], idx_v, sem_i)

    def start(r, buf):
      return pltpu.async_copy(
          yp_hbm.at[bg * BGROUP + r].at[:, pl.ds(s0, SBLK)],
          pred_b[buf], sems[buf])

    pend = [start(0, 0), start(1, 1)]

    pltpu.sync_copy(b_hbm, b_v)
    for i in range(VOCAB * VOCAB // LANES):
      bv = b_v[pl.ds(i * LANES, LANES)]
      for l in range(LANES):
        brep_v[pl.ds(l * LANE_STRIDE + i * LANES, LANES)] = bv

    lane_off = lax.iota(jnp.int32, LANES) * LANE_STRIDE
    d_idx.wait()

    n_acc = 8
    accs = tuple(jnp.zeros((LANES,), jnp.float32) for _ in range(n_acc))
    for r in range(BGROUP):
      buf = r % 2
      pend[buf].wait()
      pv = pred_b[buf]

      def sb_body(sb, carry, pv=pv, r=r):
        k = idx_v[r, pl.ds(sb * LANES, LANES)]
        k24 = k * VOCAB + lane_off
        out = list(carry)
        for v in range(VOCAB):
          data = pv[v, pl.ds(sb * LANES, LANES)]
          w = plsc.load_gather(brep_v, [k24 + v])
          out[v % n_acc] = out[v % n_acc] + w * data
        return tuple(out)

      accs = plsc.parallel_loop(0, SBLK // LANES, carry=accs)(sb_body)
      if r + 2 < BGROUP:
        pend[buf] = start(r + 2, buf)

    total = accs[0]
    for v in range(1, n_acc):
      total = total + accs[v]
    acc_v[...] = total
    pltpu.sync_copy(acc_v, out_hbm.at[pl.ds(wid * LANES, LANES)])

  return blosum_kernel


@jax.jit
def kernel(y_true, y_pred, B):
  batch, seq = y_true.shape
  yp_t = jnp.transpose(y_pred, (0, 2, 1))
  partials = _make_kernel(batch, seq)(y_true, yp_t, B.reshape(-1))
  return jnp.sum(partials)


# speedup vs baseline: 35.9286x; 1.3847x over previous
import functools

import jax
import jax.numpy as jnp
from jax import lax
from jax.experimental import pallas as pl
from jax.experimental.pallas import tpu as pltpu
from jax.experimental.pallas import tpu_sc as plsc

VOCAB = 24
LANES = 16
N_CORES = 2
N_SUBCORES = 16
N_WORKERS = N_CORES * N_SUBCORES
BGROUP = 8
KSTRIDE = 32
LANE_STRIDE = VOCAB * KSTRIDE + 1
REP_WORDS = LANE_STRIDE * LANES
SC_BATCH = 32
TC_SBLK = 2048


def _make_sc_kernel(sc_batch: int, seq: int):
  n_sblocks = N_WORKERS // (sc_batch // BGROUP)
  sblk = seq // n_sblocks
  mesh = plsc.VectorSubcoreMesh(core_axis_name="c", subcore_axis_name="s")

  scratch = (
      [pltpu.VMEM((VOCAB, sblk), jnp.float32)] * 2
      + [
          pltpu.VMEM((BGROUP, sblk), jnp.int32),
          pltpu.VMEM((VOCAB * VOCAB,), jnp.float32),
          pltpu.VMEM((REP_WORDS,), jnp.float32),
          pltpu.VMEM((LANES,), jnp.float32),
      ]
      + [pltpu.SemaphoreType.DMA] * 3
  )

  @functools.partial(
      pl.kernel,
      out_type=jax.ShapeDtypeStruct((N_WORKERS * LANES,), jnp.float32),
      mesh=mesh,
      scratch_types=scratch,
      compiler_params=pltpu.CompilerParams(
          use_tc_tiling_on_sc=True, needs_layout_passes=False),
  )
  def blosum_sc(yt_hbm, yp_hbm, b_hbm, out_hbm,
                pred0, pred1, idx_v, b_v, brep_v, acc_v,
                sem0, sem1, sem_i):
    pred_b = (pred0, pred1)
    sems = (sem0, sem1)
    cid = lax.axis_index("c")
    sid = lax.axis_index("s")
    wid = sid * N_CORES + cid
    bg = wid // n_sblocks
    s0 = (wid % n_sblocks) * sblk

    d_idx = pltpu.async_copy(
        yt_hbm.at[pl.ds(bg * BGROUP, BGROUP), pl.ds(s0, sblk)], idx_v, sem_i)

    def start(r, buf):
      return pltpu.async_copy(
          yp_hbm.at[bg * BGROUP + r].at[:, pl.ds(s0, sblk)],
          pred_b[buf], sems[buf])

    pend = [start(0, 0), start(1, 1)]

    pltpu.sync_copy(b_hbm, b_v)
    for k in range(VOCAB):
      lo = b_v[pl.ds(k * VOCAB, LANES)]
      hi = b_v[pl.ds(k * VOCAB + VOCAB - LANES, LANES)]
      for l in range(LANES):
        off = l * LANE_STRIDE + k * KSTRIDE
        brep_v[pl.ds(off, LANES)] = lo
        brep_v[pl.ds(off + VOCAB - LANES, LANES)] = hi

    lane_off = lax.iota(jnp.int32, LANES) * LANE_STRIDE
    d_idx.wait()

    n_acc = 8
    accs = tuple(jnp.zeros((LANES,), jnp.float32) for _ in range(n_acc))
    for r in range(BGROUP):
      buf = r % 2
      pend[buf].wait()
      pv = pred_b[buf]

      def sb_body(sb, carry, pv=pv, r=r):
        k = idx_v[r, pl.ds(sb * LANES, LANES)]
        koff = k * KSTRIDE + lane_off
        out = list(carry)
        for v in range(VOCAB):
          data = pv[v, pl.ds(sb * LANES, LANES)]
          w = plsc.load_gather(brep_v, [koff + v])
          out[v % n_acc] = out[v % n_acc] + w * data
        return tuple(out)

      accs = plsc.parallel_loop(0, sblk // LANES, carry=accs)(sb_body)
      if r + 2 < BGROUP:
        pend[buf] = start(r + 2, buf)

    total = accs[0]
    for v in range(1, n_acc):
      total = total + accs[v]
    acc_v[...] = total
    pltpu.sync_copy(acc_v, out_hbm.at[pl.ds(wid * LANES, LANES)])

  return blosum_sc


def _tc_body(yt_ref, yp_ref, b_ref, out_ref):
  i = pl.program_id(0)
  j = pl.program_id(1)
  iota_v = lax.broadcasted_iota(jnp.int32, (VOCAB, TC_SBLK), 0)
  s = jnp.float32(0.0)
  for r in range(BGROUP):
    k = yt_ref[r, :]
    onehot = (k[None, :] == iota_v).astype(jnp.float32)
    w = lax.dot_general(b_ref[...], onehot, (((0,), (0,)), ((), ())),
                        preferred_element_type=jnp.float32)
    s = s + jnp.sum(w * yp_ref[r])

  @pl.when((i == 0) & (j == 0))
  def _():
    out_ref[0, 0] = jnp.float32(0.0)

  out_ref[0, 0] += s


def _make_tc_kernel(tc_batch: int, seq: int, b_start: int):
  grid = (tc_batch // BGROUP, seq // TC_SBLK)
  bg0 = b_start // BGROUP
  return pl.pallas_call(
      _tc_body,
      grid=grid,
      in_specs=[
          pl.BlockSpec((BGROUP, TC_SBLK), lambda i, j: (i + bg0, j)),
          pl.BlockSpec((BGROUP, VOCAB, TC_SBLK),
                       lambda i, j: (i + bg0, 0, j)),
          pl.BlockSpec((VOCAB, VOCAB), lambda i, j: (0, 0)),
      ],
      out_specs=pl.BlockSpec(
          (1, 1), lambda i, j: (0, 0), memory_space=pltpu.SMEM),
      out_shape=jax.ShapeDtypeStruct((1, 1), jnp.float32),
      compiler_params=pltpu.CompilerParams(
          dimension_semantics=("arbitrary", "arbitrary")),
  )


@jax.jit
def kernel(y_true, y_pred, B):
  batch, seq = y_true.shape
  yp_t = jnp.transpose(y_pred, (0, 2, 1))
  sc_partials = _make_sc_kernel(SC_BATCH, seq)(y_true, yp_t, B.reshape(-1))
  tc_partial = _make_tc_kernel(batch - SC_BATCH, seq, SC_BATCH)(
      y_true, yp_t, B)
  return jnp.sum(sc_partials) + tc_partial[0, 0]
